# Initial kernel scaffold; baseline (speedup 1.0000x reference)
import jax
import jax.numpy as jnp
from jax.experimental import pallas as pl


def kernel(token_ids, a_n, b_n, W, b):
    raise NotImplementedError("write your pallas kernel here")


# speedup vs baseline: 6.1616x; 6.1616x over previous
import functools
import math

import jax
import jax.numpy as jnp
from jax import lax
from jax.experimental import pallas as pl
from jax.experimental.pallas import tpu as pltpu
from jax.experimental.pallas import tpu_sc as plsc

VOCAB = 100000
NUM_FREQ = 50
EMBED_DIM = 64
FPAD = 64
ROW_BLOCK = 2000

CHUNK = 128
GROUP = 4
SUPER = CHUNK * GROUP


def _tc_table_body(a_ref, b_ref, wt_ref, bias_ref, out_ref):
    i = pl.program_id(0)
    v = (lax.broadcasted_iota(jnp.int32, (ROW_BLOCK, FPAD), 0)
         + i * ROW_BLOCK).astype(jnp.float32)
    x = v / float(VOCAB)
    f = (lax.broadcasted_iota(jnp.int32, (ROW_BLOCK, FPAD), 1) + 1
         ).astype(jnp.float32)
    theta = 2.0 * math.pi * f * x
    emb = a_ref[...] * jnp.cos(theta) + b_ref[...] * jnp.sin(theta)
    out_ref[...] = (
        jnp.dot(emb, wt_ref[...], preferred_element_type=jnp.float32)
        + bias_ref[0:1, :]
    )


def _build_table(a_p, b_p, wt, bias_blk):
    grid = VOCAB // ROW_BLOCK
    return pl.pallas_call(
        _tc_table_body,
        grid=(grid,),
        in_specs=[
            pl.BlockSpec((ROW_BLOCK, FPAD), lambda i: (i, 0)),
            pl.BlockSpec((ROW_BLOCK, FPAD), lambda i: (i, 0)),
            pl.BlockSpec((FPAD, EMBED_DIM), lambda i: (0, 0)),
            pl.BlockSpec((8, EMBED_DIM), lambda i: (0, 0)),
        ],
        out_specs=pl.BlockSpec((ROW_BLOCK, EMBED_DIM), lambda i: (i, 0)),
        out_shape=jax.ShapeDtypeStruct((VOCAB, EMBED_DIM), jnp.float32),
    )(a_p, b_p, wt, bias_blk)


def _sc_gather(ids2d, table):
    n_rows = ids2d.shape[0]
    ntok = n_rows * CHUNK
    info = plsc.get_sparse_core_info()
    nc, ns = info.num_cores, info.num_subcores
    nw = nc * ns
    rows_per_w = n_rows // nw
    iters = rows_per_w // GROUP

    mesh = plsc.VectorSubcoreMesh(core_axis_name="c", subcore_axis_name="s")

    @functools.partial(
        pl.kernel,
        mesh=mesh,
        out_type=jax.ShapeDtypeStruct((ntok, EMBED_DIM), jnp.float32),
        scratch_types=[
            pltpu.VMEM((rows_per_w, CHUNK), jnp.int32),
            pltpu.VMEM((SUPER, EMBED_DIM), jnp.float32),
            pltpu.SemaphoreType.DMA,
        ],
        compiler_params=pltpu.CompilerParams(use_tc_tiling_on_sc=False),
    )
    def k(ids_hbm, table_hbm, out_hbm, idx_v, rows_v, gsem):
        wid = lax.axis_index("s") * nc + lax.axis_index("c")
        row0 = wid * rows_per_w
        tok0 = row0 * CHUNK
        pltpu.sync_copy(ids_hbm.at[pl.ds(row0, rows_per_w)], idx_v)

        def body(s, carry):
            copies = []
            for j in range(GROUP):
                copies.append(pltpu.async_copy(
                    table_hbm.at[idx_v.at[s * GROUP + j]],
                    rows_v.at[pl.ds(j * CHUNK, CHUNK)],
                    gsem,
                ))
            for c in copies:
                c.wait()
            pltpu.sync_copy(rows_v, out_hbm.at[pl.ds(tok0 + s * SUPER, SUPER)])
            return carry

        lax.fori_loop(0, iters, body, 0)

    return k(ids2d, table)


def kernel(token_ids, a_n, b_n, W, b):
    B, S = token_ids.shape
    a_p = jnp.pad(a_n, ((0, 0), (0, FPAD - NUM_FREQ)))
    b_p = jnp.pad(b_n, ((0, 0), (0, FPAD - NUM_FREQ)))
    wt = jnp.pad(W, ((0, 0), (0, FPAD - NUM_FREQ))).T
    bias_blk = jnp.broadcast_to(b.reshape(1, EMBED_DIM), (8, EMBED_DIM))
    table = _build_table(a_p, b_p, wt, bias_blk)
    ids2d = token_ids.reshape(-1, CHUNK).astype(jnp.int32)
    out = _sc_gather(ids2d, table)
    return out.reshape(B, S, EMBED_DIM)
